# Initial kernel scaffold; baseline (speedup 1.0000x reference)
import jax
import jax.numpy as jnp
from jax.experimental import pallas as pl


def kernel(x, edge_index, W_l0, W_r0, b0, gamma0, beta0, W_l1, W_r1, b1, gamma1, beta1):
    raise NotImplementedError("write your pallas kernel here")


# speedup vs baseline: 2.7680x; 2.7680x over previous
import functools

import jax
import jax.numpy as jnp
from jax import lax
from jax.experimental import pallas as pl
from jax.experimental.pallas import tpu as pltpu
from jax.experimental.pallas import tpu_sc as plsc

N = 10000
E = 320000
D = 128
NC, NS = 2, 16
NW = NC * NS
CH = 128
NCH = 80
GC = 16
NG = NCH // GC
EPW = NCH * CH
EPAD = NW * EPW
NPAD = 10112
RPT = NPAD // NS
CW = 16
TRASH = N
_ZCHUNKS = [128, 128, 128, 128, 120]


_MESH = plsc.VectorSubcoreMesh(
    core_axis_name="c", subcore_axis_name="s", num_cores=NC, num_subcores=NS
)


@functools.partial(
    pl.kernel,
    out_type=jax.ShapeDtypeStruct((NC * NPAD, D), jnp.float32),
    mesh=_MESH,
    scratch_types=(
        pltpu.VMEM_SHARED((NPAD, D), jnp.float32),
        pltpu.VMEM((CH,), jnp.int32),
        pltpu.VMEM((CH, D), jnp.float32),
        pltpu.SemaphoreType.DMA,
    ),
)
def _sc_cnt(dst_hbm, zrow_hbm, ones_hbm, out_hbm,
            cnt_sh, dstv, onesv, sem):
    c = lax.axis_index("c")
    s = lax.axis_index("s")
    wid = c * NS + s
    ebase = wid * EPW
    pltpu.sync_copy(zrow_hbm, onesv)
    off = 0
    for z in _ZCHUNKS:
        pltpu.sync_copy(onesv.at[pl.ds(0, z)],
                        cnt_sh.at[pl.ds(s * RPT + off, z)])
        off += z
    pltpu.sync_copy(ones_hbm, onesv)
    plsc.subcore_barrier()

    def body(g, carry):
        pltpu.sync_copy(dst_hbm.at[pl.ds(ebase + g * CH, CH)], dstv)
        pltpu.sync_copy(onesv, cnt_sh.at[dstv], add=True)
        return carry

    lax.fori_loop(0, NCH, body, 0)
    plsc.subcore_barrier()
    off = 0
    for z in _ZCHUNKS:
        pltpu.sync_copy(cnt_sh.at[pl.ds(s * RPT + off, z)],
                        onesv.at[pl.ds(0, z)])
        pltpu.sync_copy(onesv.at[pl.ds(0, z)],
                        out_hbm.at[pl.ds(c * NPAD + s * RPT + off, z)])
        off += z


@functools.partial(
    pl.kernel,
    out_type=jax.ShapeDtypeStruct((NC * NPAD, D), jnp.float32),
    mesh=_MESH,
    scratch_types=(
        pltpu.VMEM_SHARED((NPAD, D), jnp.float32),
        pltpu.VMEM((CH,), jnp.int32),
        pltpu.VMEM((CH,), jnp.int32),
        pltpu.VMEM((CH, D), jnp.float32),
        pltpu.SemaphoreType.DMA,
    ),
)
def _sc_agg(a_hbm, src_hbm, dst_hbm, zrow_hbm, out_hbm,
            acc_sh, srcv, dstv, rows, sem):
    c = lax.axis_index("c")
    s = lax.axis_index("s")
    wid = c * NS + s
    ebase = wid * EPW
    pltpu.sync_copy(zrow_hbm, rows)
    off = 0
    for z in _ZCHUNKS:
        pltpu.sync_copy(rows.at[pl.ds(0, z)],
                        acc_sh.at[pl.ds(s * RPT + off, z)])
        off += z
    plsc.subcore_barrier()

    def body(g, carry):
        pltpu.sync_copy(src_hbm.at[pl.ds(ebase + g * CH, CH)], srcv)
        pltpu.sync_copy(dst_hbm.at[pl.ds(ebase + g * CH, CH)], dstv)
        pltpu.async_copy(a_hbm.at[srcv], rows, sem).wait()
        pltpu.sync_copy(rows, acc_sh.at[dstv], add=True)
        return carry

    lax.fori_loop(0, NCH, body, 0)
    plsc.subcore_barrier()
    off = 0
    for z in _ZCHUNKS:
        pltpu.sync_copy(acc_sh.at[pl.ds(s * RPT + off, z)],
                        rows.at[pl.ds(0, z)])
        pltpu.sync_copy(rows.at[pl.ds(0, z)],
                        out_hbm.at[pl.ds(c * NPAD + s * RPT + off, z)])
        off += z


_BR = 1000


def _lin_body(h_ref, wl_ref, wr_ref, b_ref, a_ref, r_ref):
    h = h_ref[...]
    a_ref[...] = jnp.dot(h, wl_ref[...], preferred_element_type=jnp.float32)
    r_ref[...] = jnp.dot(h, wr_ref[...],
                         preferred_element_type=jnp.float32) + b_ref[...]


def _lin(h, wl, wr, b):
    return pl.pallas_call(
        _lin_body,
        grid=(N // _BR,),
        in_specs=[
            pl.BlockSpec((_BR, D), lambda i: (i, 0)),
            pl.BlockSpec((D, D), lambda i: (0, 0)),
            pl.BlockSpec((D, D), lambda i: (0, 0)),
            pl.BlockSpec((1, D), lambda i: (0, 0)),
        ],
        out_specs=[
            pl.BlockSpec((_BR, D), lambda i: (i, 0)),
            pl.BlockSpec((_BR, D), lambda i: (i, 0)),
        ],
        out_shape=[
            jax.ShapeDtypeStruct((N, D), jnp.float32),
            jax.ShapeDtypeStruct((N, D), jnp.float32),
        ],
    )(h, wl, wr, b.reshape(1, D))


def _norm_block(p0, p1, c0, c1, r, g, bt):
    cnt = c0[0, :, 0:1] + c1[0, :, 0:1]
    h = (p0[0] + p1[0]) / jnp.clip(cnt, 1.0, None) + r
    mu = jnp.mean(h, axis=1, keepdims=True)
    var = jnp.mean((h - mu) ** 2, axis=1, keepdims=True)
    h = (h - mu) * lax.rsqrt(var + 1e-5) * g + bt
    return jnp.maximum(h, 0.0)


def _comblin_body(p_ref, p_ref2, c_ref, c_ref2, r_ref, g_ref, bt_ref,
                  wl_ref, wr_ref, b_ref, a_ref, rn_ref):
    h = _norm_block(p_ref[...], p_ref2[...], c_ref[...], c_ref2[...],
                    r_ref[...], g_ref[...], bt_ref[...])
    a_ref[...] = jnp.dot(h, wl_ref[...], preferred_element_type=jnp.float32)
    rn_ref[...] = jnp.dot(h, wr_ref[...],
                          preferred_element_type=jnp.float32) + b_ref[...]


def _comblin(parts, cnts, r, gamma, beta, wl, wr, b):
    return pl.pallas_call(
        _comblin_body,
        grid=(N // _BR,),
        in_specs=[
            pl.BlockSpec((1, _BR, D), lambda i: (0, i, 0)),
            pl.BlockSpec((1, _BR, D), lambda i: (1, i, 0)),
            pl.BlockSpec((1, _BR, CW), lambda i: (0, i, 0)),
            pl.BlockSpec((1, _BR, CW), lambda i: (1, i, 0)),
            pl.BlockSpec((_BR, D), lambda i: (i, 0)),
            pl.BlockSpec((1, D), lambda i: (0, 0)),
            pl.BlockSpec((1, D), lambda i: (0, 0)),
            pl.BlockSpec((D, D), lambda i: (0, 0)),
            pl.BlockSpec((D, D), lambda i: (0, 0)),
            pl.BlockSpec((1, D), lambda i: (0, 0)),
        ],
        out_specs=[
            pl.BlockSpec((_BR, D), lambda i: (i, 0)),
            pl.BlockSpec((_BR, D), lambda i: (i, 0)),
        ],
        out_shape=[
            jax.ShapeDtypeStruct((N, D), jnp.float32),
            jax.ShapeDtypeStruct((N, D), jnp.float32),
        ],
    )(parts, parts, cnts, cnts, r, gamma.reshape(1, D), beta.reshape(1, D),
      wl, wr, b.reshape(1, D))


def _comb_body(p_ref, p_ref2, c_ref, c_ref2, r_ref, g_ref, bt_ref, o_ref):
    o_ref[...] = _norm_block(p_ref[...], p_ref2[...], c_ref[...], c_ref2[...],
                             r_ref[...], g_ref[...], bt_ref[...])


def _comb(parts, cnts, r, gamma, beta):
    return pl.pallas_call(
        _comb_body,
        grid=(N // _BR,),
        in_specs=[
            pl.BlockSpec((1, _BR, D), lambda i: (0, i, 0)),
            pl.BlockSpec((1, _BR, D), lambda i: (1, i, 0)),
            pl.BlockSpec((1, _BR, CW), lambda i: (0, i, 0)),
            pl.BlockSpec((1, _BR, CW), lambda i: (1, i, 0)),
            pl.BlockSpec((_BR, D), lambda i: (i, 0)),
            pl.BlockSpec((1, D), lambda i: (0, 0)),
            pl.BlockSpec((1, D), lambda i: (0, 0)),
        ],
        out_specs=pl.BlockSpec((_BR, D), lambda i: (i, 0)),
        out_shape=jax.ShapeDtypeStruct((N, D), jnp.float32),
    )(parts, parts, cnts, cnts, r, gamma.reshape(1, D), beta.reshape(1, D))


def kernel(x, edge_index, W_l0, W_r0, b0, gamma0, beta0,
           W_l1, W_r1, b1, gamma1, beta1):
    src = edge_index[0].astype(jnp.int32)
    dst = edge_index[1].astype(jnp.int32)
    pad = EPAD - E
    src_p = jnp.concatenate([src, jnp.zeros((pad,), jnp.int32)])
    dst_p = jnp.concatenate([dst, jnp.full((pad,), TRASH, jnp.int32)])

    zrow = jnp.zeros((CH, D), jnp.float32)

    a0, r0 = _lin(x, W_l0, W_r0, b0)
    ones = jnp.ones((CH, D), jnp.float32)
    cnts = _sc_cnt(dst_p, zrow, ones).reshape(NC, NPAD, D)[:, :, :CW]

    parts0 = _sc_agg(a0, src_p, dst_p, zrow).reshape(NC, NPAD, D)
    a1, r1 = _comblin(parts0, cnts, r0, gamma0, beta0, W_l1, W_r1, b1)
    parts1 = _sc_agg(a1, src_p, dst_p, zrow).reshape(NC, NPAD, D)
    return _comb(parts1, cnts, r1, gamma1, beta1)


# speedup vs baseline: 5.6905x; 2.0558x over previous
import functools

import jax
import jax.numpy as jnp
from jax import lax
from jax.experimental import pallas as pl
from jax.experimental.pallas import tpu as pltpu
from jax.experimental.pallas import tpu_sc as plsc

N = 10000
E = 320000
D = 128
NC, NS = 2, 16
NW = NC * NS
CH = 128
NCH = 80
GC = 16
NG = NCH // GC
EPW = NCH * CH
EPAD = NW * EPW
NPAD = 10112
RPT = NPAD // NS
CW = 16
TRASH = N
_ZCHUNKS = [128, 128, 128, 128, 120]


_MESH = plsc.VectorSubcoreMesh(
    core_axis_name="c", subcore_axis_name="s", num_cores=NC, num_subcores=NS
)


@functools.partial(
    pl.kernel,
    out_type=jax.ShapeDtypeStruct((NC * NPAD, D), jnp.float32),
    mesh=_MESH,
    scratch_types=(
        pltpu.VMEM_SHARED((NPAD, D), jnp.float32),
        pltpu.VMEM((CH,), jnp.int32),
        pltpu.VMEM((CH, D), jnp.float32),
        pltpu.SemaphoreType.DMA,
    ),
)
def _sc_cnt(dst_hbm, zrow_hbm, ones_hbm, out_hbm,
            cnt_sh, dstv, onesv, sem):
    c = lax.axis_index("c")
    s = lax.axis_index("s")
    wid = c * NS + s
    ebase = wid * EPW
    pltpu.sync_copy(zrow_hbm, onesv)
    off = 0
    for z in _ZCHUNKS:
        pltpu.sync_copy(onesv.at[pl.ds(0, z)],
                        cnt_sh.at[pl.ds(s * RPT + off, z)])
        off += z
    pltpu.sync_copy(ones_hbm, onesv)
    plsc.subcore_barrier()

    def body(g, carry):
        pltpu.sync_copy(dst_hbm.at[pl.ds(ebase + g * CH, CH)], dstv)
        pltpu.sync_copy(onesv, cnt_sh.at[dstv], add=True)
        return carry

    lax.fori_loop(0, NCH, body, 0)
    plsc.subcore_barrier()
    off = 0
    for z in _ZCHUNKS:
        pltpu.sync_copy(cnt_sh.at[pl.ds(s * RPT + off, z)],
                        onesv.at[pl.ds(0, z)])
        pltpu.sync_copy(onesv.at[pl.ds(0, z)],
                        out_hbm.at[pl.ds(c * NPAD + s * RPT + off, z)])
        off += z


@functools.partial(
    pl.kernel,
    out_type=jax.ShapeDtypeStruct((NC * NPAD, D), jnp.float32),
    mesh=_MESH,
    scratch_types=(
        pltpu.VMEM_SHARED((NPAD, D), jnp.float32),
        pltpu.VMEM((CH,), jnp.int32),
        pltpu.VMEM((CH,), jnp.int32),
        pltpu.VMEM((CH, D), jnp.float32),
        pltpu.SemaphoreType.DMA,
    ),
)
def _sc_agg(a_hbm, src_hbm, dst_hbm, zrow_hbm, out_hbm,
            acc_sh, srcv, dstv, rows, sem):
    c = lax.axis_index("c")
    s = lax.axis_index("s")
    wid = c * NS + s
    ebase = wid * EPW
    pltpu.sync_copy(zrow_hbm, rows)
    off = 0
    for z in _ZCHUNKS:
        pltpu.sync_copy(rows.at[pl.ds(0, z)],
                        acc_sh.at[pl.ds(s * RPT + off, z)])
        off += z
    plsc.subcore_barrier()

    def body(g, carry):
        pltpu.sync_copy(src_hbm.at[pl.ds(ebase + g * CH, CH)], srcv)
        pltpu.sync_copy(dst_hbm.at[pl.ds(ebase + g * CH, CH)], dstv)
        pltpu.async_copy(a_hbm.at[srcv], rows, sem).wait()
        pltpu.sync_copy(rows, acc_sh.at[dstv], add=True)
        return carry

    lax.fori_loop(0, NCH, body, 0)
    plsc.subcore_barrier()
    off = 0
    for z in _ZCHUNKS:
        pltpu.sync_copy(acc_sh.at[pl.ds(s * RPT + off, z)],
                        rows.at[pl.ds(0, z)])
        pltpu.sync_copy(rows.at[pl.ds(0, z)],
                        out_hbm.at[pl.ds(c * NPAD + s * RPT + off, z)])
        off += z


_BR = 1000


def _lin_body(h_ref, wl_ref, wr_ref, b_ref, a_ref, r_ref):
    h = h_ref[...]
    a_ref[...] = jnp.dot(h, wl_ref[...], preferred_element_type=jnp.float32)
    r_ref[...] = jnp.dot(h, wr_ref[...],
                         preferred_element_type=jnp.float32) + b_ref[...]


def _lin(h, wl, wr, b):
    return pl.pallas_call(
        _lin_body,
        grid=(N // _BR,),
        in_specs=[
            pl.BlockSpec((_BR, D), lambda i: (i, 0)),
            pl.BlockSpec((D, D), lambda i: (0, 0)),
            pl.BlockSpec((D, D), lambda i: (0, 0)),
            pl.BlockSpec((1, D), lambda i: (0, 0)),
        ],
        out_specs=[
            pl.BlockSpec((_BR, D), lambda i: (i, 0)),
            pl.BlockSpec((_BR, D), lambda i: (i, 0)),
        ],
        out_shape=[
            jax.ShapeDtypeStruct((N, D), jnp.float32),
            jax.ShapeDtypeStruct((N, D), jnp.float32),
        ],
    )(h, wl, wr, b.reshape(1, D))


def _norm_block(p0, p1, c0, c1, r, g, bt):
    cnt = c0[0, :, 0:1] + c1[0, :, 0:1]
    h = (p0[0] + p1[0]) / jnp.clip(cnt, 1.0, None) + r
    mu = jnp.mean(h, axis=1, keepdims=True)
    var = jnp.mean((h - mu) ** 2, axis=1, keepdims=True)
    h = (h - mu) * lax.rsqrt(var + 1e-5) * g + bt
    return jnp.maximum(h, 0.0)


def _comblin_body(p_ref, p_ref2, c_ref, c_ref2, r_ref, g_ref, bt_ref,
                  wl_ref, wr_ref, b_ref, a_ref, rn_ref):
    h = _norm_block(p_ref[...], p_ref2[...], c_ref[...], c_ref2[...],
                    r_ref[...], g_ref[...], bt_ref[...])
    a_ref[...] = jnp.dot(h, wl_ref[...], preferred_element_type=jnp.float32)
    rn_ref[...] = jnp.dot(h, wr_ref[...],
                          preferred_element_type=jnp.float32) + b_ref[...]


def _comblin(parts, cnts, r, gamma, beta, wl, wr, b):
    return pl.pallas_call(
        _comblin_body,
        grid=(N // _BR,),
        in_specs=[
            pl.BlockSpec((1, _BR, D), lambda i: (0, i, 0)),
            pl.BlockSpec((1, _BR, D), lambda i: (1, i, 0)),
            pl.BlockSpec((1, _BR, CW), lambda i: (0, i, 0)),
            pl.BlockSpec((1, _BR, CW), lambda i: (1, i, 0)),
            pl.BlockSpec((_BR, D), lambda i: (i, 0)),
            pl.BlockSpec((1, D), lambda i: (0, 0)),
            pl.BlockSpec((1, D), lambda i: (0, 0)),
            pl.BlockSpec((D, D), lambda i: (0, 0)),
            pl.BlockSpec((D, D), lambda i: (0, 0)),
            pl.BlockSpec((1, D), lambda i: (0, 0)),
        ],
        out_specs=[
            pl.BlockSpec((_BR, D), lambda i: (i, 0)),
            pl.BlockSpec((_BR, D), lambda i: (i, 0)),
        ],
        out_shape=[
            jax.ShapeDtypeStruct((N, D), jnp.float32),
            jax.ShapeDtypeStruct((N, D), jnp.float32),
        ],
    )(parts, parts, cnts, cnts, r, gamma.reshape(1, D), beta.reshape(1, D),
      wl, wr, b.reshape(1, D))


def _comb_body(p_ref, p_ref2, c_ref, c_ref2, r_ref, g_ref, bt_ref, o_ref):
    o_ref[...] = _norm_block(p_ref[...], p_ref2[...], c_ref[...], c_ref2[...],
                             r_ref[...], g_ref[...], bt_ref[...])


def _comb(parts, cnts, r, gamma, beta):
    return pl.pallas_call(
        _comb_body,
        grid=(N // _BR,),
        in_specs=[
            pl.BlockSpec((1, _BR, D), lambda i: (0, i, 0)),
            pl.BlockSpec((1, _BR, D), lambda i: (1, i, 0)),
            pl.BlockSpec((1, _BR, CW), lambda i: (0, i, 0)),
            pl.BlockSpec((1, _BR, CW), lambda i: (1, i, 0)),
            pl.BlockSpec((_BR, D), lambda i: (i, 0)),
            pl.BlockSpec((1, D), lambda i: (0, 0)),
            pl.BlockSpec((1, D), lambda i: (0, 0)),
        ],
        out_specs=pl.BlockSpec((_BR, D), lambda i: (i, 0)),
        out_shape=jax.ShapeDtypeStruct((N, D), jnp.float32),
    )(parts, parts, cnts, cnts, r, gamma.reshape(1, D), beta.reshape(1, D))


def kernel(x, edge_index, W_l0, W_r0, b0, gamma0, beta0,
           W_l1, W_r1, b1, gamma1, beta1):
    src = edge_index[0].astype(jnp.int32)
    dst = edge_index[1].astype(jnp.int32)
    pad = EPAD - E
    pk = jnp.arange(pad, dtype=jnp.int32)
    src_p = jnp.concatenate([src, pk % N])
    dst_p = jnp.concatenate([dst, TRASH + pk % (NPAD - N)])

    zrow = jnp.zeros((CH, D), jnp.float32)

    a0, r0 = _lin(x, W_l0, W_r0, b0)
    ones = jnp.ones((CH, D), jnp.float32)
    cnts = _sc_cnt(dst_p, zrow, ones).reshape(NC, NPAD, D)[:, :, :CW]

    parts0 = _sc_agg(a0, src_p, dst_p, zrow).reshape(NC, NPAD, D)
    a1, r1 = _comblin(parts0, cnts, r0, gamma0, beta0, W_l1, W_r1, b1)
    parts1 = _sc_agg(a1, src_p, dst_p, zrow).reshape(NC, NPAD, D)
    return _comb(parts1, cnts, r1, gamma1, beta1)


# speedup vs baseline: 8.0248x; 1.4102x over previous
import functools

import jax
import jax.numpy as jnp
from jax import lax
from jax.experimental import pallas as pl
from jax.experimental.pallas import tpu as pltpu
from jax.experimental.pallas import tpu_sc as plsc

N = 10000
E = 320000
D = 128
NC, NS = 2, 16
NW = NC * NS
CH = 128
NCH = 80
GC = 16
NG = NCH // GC
EPW = NCH * CH
EPAD = NW * EPW
NPAD = 10112
RPT = NPAD // NS
CW = 16
TRASH = N
_ZCHUNKS = [128, 128, 128, 128, 120]


_MESH = plsc.VectorSubcoreMesh(
    core_axis_name="c", subcore_axis_name="s", num_cores=NC, num_subcores=NS
)


@functools.partial(
    pl.kernel,
    out_type=jax.ShapeDtypeStruct((NC * NPAD, D), jnp.float32),
    mesh=_MESH,
    scratch_types=(
        pltpu.VMEM_SHARED((NPAD, D), jnp.float32),
        pltpu.VMEM((CH,), jnp.int32),
        pltpu.VMEM((CH, D), jnp.float32),
        pltpu.SemaphoreType.DMA,
    ),
)
def _sc_cnt(dst_hbm, zrow_hbm, ones_hbm, out_hbm,
            cnt_sh, dstv, onesv, sem):
    c = lax.axis_index("c")
    s = lax.axis_index("s")
    wid = c * NS + s
    ebase = wid * EPW
    pltpu.sync_copy(zrow_hbm, onesv)
    off = 0
    for z in _ZCHUNKS:
        pltpu.sync_copy(onesv.at[pl.ds(0, z)],
                        cnt_sh.at[pl.ds(s * RPT + off, z)])
        off += z
    pltpu.sync_copy(ones_hbm, onesv)
    plsc.subcore_barrier()

    def body(g, carry):
        pltpu.sync_copy(dst_hbm.at[pl.ds(ebase + g * CH, CH)], dstv)
        pltpu.sync_copy(onesv, cnt_sh.at[dstv], add=True)
        return carry

    lax.fori_loop(0, NCH, body, 0)
    plsc.subcore_barrier()
    off = 0
    for z in _ZCHUNKS:
        pltpu.sync_copy(cnt_sh.at[pl.ds(s * RPT + off, z)],
                        onesv.at[pl.ds(0, z)])
        pltpu.sync_copy(onesv.at[pl.ds(0, z)],
                        out_hbm.at[pl.ds(c * NPAD + s * RPT + off, z)])
        off += z


@functools.partial(
    pl.kernel,
    out_type=jax.ShapeDtypeStruct((NC * NPAD, D), jnp.float32),
    mesh=_MESH,
    scratch_types=(
        pltpu.VMEM_SHARED((NPAD, D), jnp.float32),
        pltpu.VMEM((CH,), jnp.int32),
        pltpu.VMEM((CH,), jnp.int32),
        pltpu.VMEM((CH,), jnp.int32),
        pltpu.VMEM((CH,), jnp.int32),
        pltpu.VMEM((CH, D), jnp.float32),
        pltpu.VMEM((CH, D), jnp.float32),
        pltpu.SemaphoreType.DMA,
        pltpu.SemaphoreType.DMA,
    ),
)
def _sc_agg(a_hbm, src_hbm, dst_hbm, zrow_hbm, out_hbm,
            acc_sh, src0, src1, dst0, dst1, rows0, rows1, sem0, sem1):
    c = lax.axis_index("c")
    s = lax.axis_index("s")
    wid = c * NS + s
    ebase = wid * EPW
    pltpu.sync_copy(zrow_hbm, rows0)
    off = 0
    for z in _ZCHUNKS:
        pltpu.sync_copy(rows0.at[pl.ds(0, z)],
                        acc_sh.at[pl.ds(s * RPT + off, z)])
        off += z
    plsc.subcore_barrier()

    pltpu.sync_copy(src_hbm.at[pl.ds(ebase, CH)], src0)
    pltpu.sync_copy(dst_hbm.at[pl.ds(ebase, CH)], dst0)
    pltpu.async_copy(a_hbm.at[src0], rows0, sem0)

    def body(g2, carry):
        g = g2 * 2
        pltpu.sync_copy(src_hbm.at[pl.ds(ebase + (g + 1) * CH, CH)], src1)
        pltpu.sync_copy(dst_hbm.at[pl.ds(ebase + (g + 1) * CH, CH)], dst1)
        pltpu.async_copy(a_hbm.at[src1], rows1, sem1)
        pltpu.make_async_copy(a_hbm.at[src0], rows0, sem0).wait()
        pltpu.sync_copy(rows0, acc_sh.at[dst0], add=True)

        @pl.when(g + 2 < NCH)
        def _():
            pltpu.sync_copy(src_hbm.at[pl.ds(ebase + (g + 2) * CH, CH)], src0)
            pltpu.sync_copy(dst_hbm.at[pl.ds(ebase + (g + 2) * CH, CH)], dst0)
            pltpu.async_copy(a_hbm.at[src0], rows0, sem0)

        pltpu.make_async_copy(a_hbm.at[src1], rows1, sem1).wait()
        pltpu.sync_copy(rows1, acc_sh.at[dst1], add=True)
        return carry

    lax.fori_loop(0, NCH // 2, body, 0)
    plsc.subcore_barrier()
    off = 0
    for z in _ZCHUNKS:
        pltpu.sync_copy(acc_sh.at[pl.ds(s * RPT + off, z)],
                        rows0.at[pl.ds(0, z)])
        pltpu.sync_copy(rows0.at[pl.ds(0, z)],
                        out_hbm.at[pl.ds(c * NPAD + s * RPT + off, z)])
        off += z


_BR = 1000


def _lin_body(h_ref, wl_ref, wr_ref, b_ref, a_ref, r_ref):
    h = h_ref[...]
    a_ref[...] = jnp.dot(h, wl_ref[...], preferred_element_type=jnp.float32)
    r_ref[...] = jnp.dot(h, wr_ref[...],
                         preferred_element_type=jnp.float32) + b_ref[...]


def _lin(h, wl, wr, b):
    return pl.pallas_call(
        _lin_body,
        grid=(N // _BR,),
        in_specs=[
            pl.BlockSpec((_BR, D), lambda i: (i, 0)),
            pl.BlockSpec((D, D), lambda i: (0, 0)),
            pl.BlockSpec((D, D), lambda i: (0, 0)),
            pl.BlockSpec((1, D), lambda i: (0, 0)),
        ],
        out_specs=[
            pl.BlockSpec((_BR, D), lambda i: (i, 0)),
            pl.BlockSpec((_BR, D), lambda i: (i, 0)),
        ],
        out_shape=[
            jax.ShapeDtypeStruct((N, D), jnp.float32),
            jax.ShapeDtypeStruct((N, D), jnp.float32),
        ],
    )(h, wl, wr, b.reshape(1, D))


def _norm_block(p0, p1, c0, c1, r, g, bt):
    cnt = c0[0, :, 0:1] + c1[0, :, 0:1]
    h = (p0[0] + p1[0]) / jnp.clip(cnt, 1.0, None) + r
    mu = jnp.mean(h, axis=1, keepdims=True)
    var = jnp.mean((h - mu) ** 2, axis=1, keepdims=True)
    h = (h - mu) * lax.rsqrt(var + 1e-5) * g + bt
    return jnp.maximum(h, 0.0)


def _comblin_body(p_ref, p_ref2, c_ref, c_ref2, r_ref, g_ref, bt_ref,
                  wl_ref, wr_ref, b_ref, a_ref, rn_ref):
    h = _norm_block(p_ref[...], p_ref2[...], c_ref[...], c_ref2[...],
                    r_ref[...], g_ref[...], bt_ref[...])
    a_ref[...] = jnp.dot(h, wl_ref[...], preferred_element_type=jnp.float32)
    rn_ref[...] = jnp.dot(h, wr_ref[...],
                          preferred_element_type=jnp.float32) + b_ref[...]


def _comblin(parts, cnts, r, gamma, beta, wl, wr, b):
    return pl.pallas_call(
        _comblin_body,
        grid=(N // _BR,),
        in_specs=[
            pl.BlockSpec((1, _BR, D), lambda i: (0, i, 0)),
            pl.BlockSpec((1, _BR, D), lambda i: (1, i, 0)),
            pl.BlockSpec((1, _BR, CW), lambda i: (0, i, 0)),
            pl.BlockSpec((1, _BR, CW), lambda i: (1, i, 0)),
            pl.BlockSpec((_BR, D), lambda i: (i, 0)),
            pl.BlockSpec((1, D), lambda i: (0, 0)),
            pl.BlockSpec((1, D), lambda i: (0, 0)),
            pl.BlockSpec((D, D), lambda i: (0, 0)),
            pl.BlockSpec((D, D), lambda i: (0, 0)),
            pl.BlockSpec((1, D), lambda i: (0, 0)),
        ],
        out_specs=[
            pl.BlockSpec((_BR, D), lambda i: (i, 0)),
            pl.BlockSpec((_BR, D), lambda i: (i, 0)),
        ],
        out_shape=[
            jax.ShapeDtypeStruct((N, D), jnp.float32),
            jax.ShapeDtypeStruct((N, D), jnp.float32),
        ],
    )(parts, parts, cnts, cnts, r, gamma.reshape(1, D), beta.reshape(1, D),
      wl, wr, b.reshape(1, D))


def _comb_body(p_ref, p_ref2, c_ref, c_ref2, r_ref, g_ref, bt_ref, o_ref):
    o_ref[...] = _norm_block(p_ref[...], p_ref2[...], c_ref[...], c_ref2[...],
                             r_ref[...], g_ref[...], bt_ref[...])


def _comb(parts, cnts, r, gamma, beta):
    return pl.pallas_call(
        _comb_body,
        grid=(N // _BR,),
        in_specs=[
            pl.BlockSpec((1, _BR, D), lambda i: (0, i, 0)),
            pl.BlockSpec((1, _BR, D), lambda i: (1, i, 0)),
            pl.BlockSpec((1, _BR, CW), lambda i: (0, i, 0)),
            pl.BlockSpec((1, _BR, CW), lambda i: (1, i, 0)),
            pl.BlockSpec((_BR, D), lambda i: (i, 0)),
            pl.BlockSpec((1, D), lambda i: (0, 0)),
            pl.BlockSpec((1, D), lambda i: (0, 0)),
        ],
        out_specs=pl.BlockSpec((_BR, D), lambda i: (i, 0)),
        out_shape=jax.ShapeDtypeStruct((N, D), jnp.float32),
    )(parts, parts, cnts, cnts, r, gamma.reshape(1, D), beta.reshape(1, D))


def kernel(x, edge_index, W_l0, W_r0, b0, gamma0, beta0,
           W_l1, W_r1, b1, gamma1, beta1):
    src = edge_index[0].astype(jnp.int32)
    dst = edge_index[1].astype(jnp.int32)
    pad = EPAD - E
    pk = jnp.arange(pad, dtype=jnp.int32)
    src_p = jnp.concatenate([src, pk % N])
    dst_p = jnp.concatenate([dst, TRASH + pk % (NPAD - N)])

    zrow = jnp.zeros((CH, D), jnp.float32)

    a0, r0 = _lin(x, W_l0, W_r0, b0)
    ones = jnp.ones((CH, D), jnp.float32)
    cnts = _sc_cnt(dst_p, zrow, ones).reshape(NC, NPAD, D)[:, :, :CW]

    parts0 = _sc_agg(a0, src_p, dst_p, zrow).reshape(NC, NPAD, D)
    a1, r1 = _comblin(parts0, cnts, r0, gamma0, beta0, W_l1, W_r1, b1)
    parts1 = _sc_agg(a1, src_p, dst_p, zrow).reshape(NC, NPAD, D)
    return _comb(parts1, cnts, r1, gamma1, beta1)


# speedup vs baseline: 8.4066x; 1.0476x over previous
import functools

import jax
import jax.numpy as jnp
from jax import lax
from jax.experimental import pallas as pl
from jax.experimental.pallas import tpu as pltpu
from jax.experimental.pallas import tpu_sc as plsc

N = 10000
E = 320000
D = 128
NC, NS = 2, 16
NW = NC * NS
CH = 128
NCH = 80
GC = 16
NG = NCH // GC
EPW = NCH * CH
EPAD = NW * EPW
NPAD = 10112
RPT = NPAD // NS
CW = 16
TRASH = N
_ZCHUNKS = [128, 128, 128, 128, 120]


_MESH = plsc.VectorSubcoreMesh(
    core_axis_name="c", subcore_axis_name="s", num_cores=NC, num_subcores=NS
)


@functools.partial(
    pl.kernel,
    out_type=jax.ShapeDtypeStruct((NC * NPAD, D), jnp.float32),
    mesh=_MESH,
    scratch_types=(
        pltpu.VMEM_SHARED((NPAD, D), jnp.float32),
        pltpu.VMEM((CH,), jnp.int32),
        pltpu.VMEM((CH, D), jnp.float32),
        pltpu.SemaphoreType.DMA,
    ),
)
def _sc_cnt(dst_hbm, zrow_hbm, ones_hbm, out_hbm,
            cnt_sh, dstv, onesv, sem):
    c = lax.axis_index("c")
    s = lax.axis_index("s")
    wid = c * NS + s
    ebase = wid * EPW
    pltpu.sync_copy(zrow_hbm, onesv)
    off = 0
    for z in _ZCHUNKS:
        pltpu.sync_copy(onesv.at[pl.ds(0, z)],
                        cnt_sh.at[pl.ds(s * RPT + off, z)])
        off += z
    pltpu.sync_copy(ones_hbm, onesv)
    plsc.subcore_barrier()

    def body(g, carry):
        pltpu.sync_copy(dst_hbm.at[pl.ds(ebase + g * CH, CH)], dstv)
        pltpu.sync_copy(onesv, cnt_sh.at[dstv], add=True)
        return carry

    lax.fori_loop(0, NCH, body, 0)
    plsc.subcore_barrier()
    off = 0
    for z in _ZCHUNKS:
        pltpu.sync_copy(cnt_sh.at[pl.ds(s * RPT + off, z)],
                        onesv.at[pl.ds(0, z)])
        pltpu.sync_copy(onesv.at[pl.ds(0, z)],
                        out_hbm.at[pl.ds(c * NPAD + s * RPT + off, z)])
        off += z


@functools.partial(
    pl.kernel,
    out_type=jax.ShapeDtypeStruct((NC * NPAD, D), jnp.float32),
    mesh=_MESH,
    scratch_types=(
        pltpu.VMEM_SHARED((NPAD, D), jnp.float32),
        pltpu.VMEM((CH,), jnp.int32),
        pltpu.VMEM((CH,), jnp.int32),
        pltpu.VMEM((CH,), jnp.int32),
        pltpu.VMEM((CH,), jnp.int32),
        pltpu.VMEM((CH, D), jnp.float32),
        pltpu.VMEM((CH, D), jnp.float32),
        pltpu.SemaphoreType.DMA,
        pltpu.SemaphoreType.DMA,
        pltpu.SemaphoreType.DMA,
        pltpu.SemaphoreType.DMA,
    ),
)
def _sc_agg(a_hbm, src_hbm, dst_hbm, zrow_hbm, out_hbm,
            acc_sh, src0, src1, dst0, dst1, rows0, rows1,
            semg0, semg1, semi0, semi1):
    c = lax.axis_index("c")
    s = lax.axis_index("s")
    wid = c * NS + s
    ebase = wid * EPW
    pltpu.sync_copy(zrow_hbm, rows0)
    off = 0
    for z in _ZCHUNKS:
        pltpu.sync_copy(rows0.at[pl.ds(0, z)],
                        acc_sh.at[pl.ds(s * RPT + off, z)])
        off += z
    plsc.subcore_barrier()

    def stage(g, sv, dv, sem):
        pltpu.async_copy(src_hbm.at[pl.ds(ebase + g * CH, CH)], sv, sem)
        pltpu.async_copy(dst_hbm.at[pl.ds(ebase + g * CH, CH)], dv, sem)

    def stage_wait(g, sv, dv, sem):
        pltpu.make_async_copy(src_hbm.at[pl.ds(ebase + g * CH, CH)], sv,
                              sem).wait()
        pltpu.make_async_copy(dst_hbm.at[pl.ds(ebase + g * CH, CH)], dv,
                              sem).wait()

    pltpu.sync_copy(src_hbm.at[pl.ds(ebase, CH)], src0)
    pltpu.sync_copy(dst_hbm.at[pl.ds(ebase, CH)], dst0)
    pltpu.async_copy(a_hbm.at[src0], rows0, semg0)
    stage(1, src1, dst1, semi1)

    def body(g2, carry):
        g = g2 * 2
        stage_wait(g + 1, src1, dst1, semi1)
        pltpu.async_copy(a_hbm.at[src1], rows1, semg1)
        pltpu.make_async_copy(a_hbm.at[src0], rows0, semg0).wait()
        pltpu.sync_copy(rows0, acc_sh.at[dst0], add=True)

        @pl.when(g + 2 < NCH)
        def _():
            stage(g + 2, src0, dst0, semi0)

        pltpu.make_async_copy(a_hbm.at[src1], rows1, semg1).wait()
        pltpu.sync_copy(rows1, acc_sh.at[dst1], add=True)

        @pl.when(g + 3 < NCH)
        def _():
            stage(g + 3, src1, dst1, semi1)

        @pl.when(g + 2 < NCH)
        def _():
            stage_wait(g + 2, src0, dst0, semi0)
            pltpu.async_copy(a_hbm.at[src0], rows0, semg0)

        return carry

    lax.fori_loop(0, NCH // 2, body, 0)
    plsc.subcore_barrier()
    off = 0
    for z in _ZCHUNKS:
        pltpu.sync_copy(acc_sh.at[pl.ds(s * RPT + off, z)],
                        rows0.at[pl.ds(0, z)])
        pltpu.sync_copy(rows0.at[pl.ds(0, z)],
                        out_hbm.at[pl.ds(c * NPAD + s * RPT + off, z)])
        off += z


_BR = 1000


def _lin_body(h_ref, wl_ref, wr_ref, b_ref, a_ref, r_ref):
    h = h_ref[...]
    a_ref[...] = jnp.dot(h, wl_ref[...], preferred_element_type=jnp.float32)
    r_ref[...] = jnp.dot(h, wr_ref[...],
                         preferred_element_type=jnp.float32) + b_ref[...]


def _lin(h, wl, wr, b):
    return pl.pallas_call(
        _lin_body,
        grid=(N // _BR,),
        in_specs=[
            pl.BlockSpec((_BR, D), lambda i: (i, 0)),
            pl.BlockSpec((D, D), lambda i: (0, 0)),
            pl.BlockSpec((D, D), lambda i: (0, 0)),
            pl.BlockSpec((1, D), lambda i: (0, 0)),
        ],
        out_specs=[
            pl.BlockSpec((_BR, D), lambda i: (i, 0)),
            pl.BlockSpec((_BR, D), lambda i: (i, 0)),
        ],
        out_shape=[
            jax.ShapeDtypeStruct((N, D), jnp.float32),
            jax.ShapeDtypeStruct((N, D), jnp.float32),
        ],
    )(h, wl, wr, b.reshape(1, D))


def _norm_block(p0, p1, c0, c1, r, g, bt):
    cnt = c0[0, :, 0:1] + c1[0, :, 0:1]
    h = (p0[0] + p1[0]) / jnp.clip(cnt, 1.0, None) + r
    mu = jnp.mean(h, axis=1, keepdims=True)
    var = jnp.mean((h - mu) ** 2, axis=1, keepdims=True)
    h = (h - mu) * lax.rsqrt(var + 1e-5) * g + bt
    return jnp.maximum(h, 0.0)


def _comblin_body(p_ref, p_ref2, c_ref, c_ref2, r_ref, g_ref, bt_ref,
                  wl_ref, wr_ref, b_ref, a_ref, rn_ref):
    h = _norm_block(p_ref[...], p_ref2[...], c_ref[...], c_ref2[...],
                    r_ref[...], g_ref[...], bt_ref[...])
    a_ref[...] = jnp.dot(h, wl_ref[...], preferred_element_type=jnp.float32)
    rn_ref[...] = jnp.dot(h, wr_ref[...],
                          preferred_element_type=jnp.float32) + b_ref[...]


def _comblin(parts, cnts, r, gamma, beta, wl, wr, b):
    return pl.pallas_call(
        _comblin_body,
        grid=(N // _BR,),
        in_specs=[
            pl.BlockSpec((1, _BR, D), lambda i: (0, i, 0)),
            pl.BlockSpec((1, _BR, D), lambda i: (1, i, 0)),
            pl.BlockSpec((1, _BR, CW), lambda i: (0, i, 0)),
            pl.BlockSpec((1, _BR, CW), lambda i: (1, i, 0)),
            pl.BlockSpec((_BR, D), lambda i: (i, 0)),
            pl.BlockSpec((1, D), lambda i: (0, 0)),
            pl.BlockSpec((1, D), lambda i: (0, 0)),
            pl.BlockSpec((D, D), lambda i: (0, 0)),
            pl.BlockSpec((D, D), lambda i: (0, 0)),
            pl.BlockSpec((1, D), lambda i: (0, 0)),
        ],
        out_specs=[
            pl.BlockSpec((_BR, D), lambda i: (i, 0)),
            pl.BlockSpec((_BR, D), lambda i: (i, 0)),
        ],
        out_shape=[
            jax.ShapeDtypeStruct((N, D), jnp.float32),
            jax.ShapeDtypeStruct((N, D), jnp.float32),
        ],
    )(parts, parts, cnts, cnts, r, gamma.reshape(1, D), beta.reshape(1, D),
      wl, wr, b.reshape(1, D))


def _comb_body(p_ref, p_ref2, c_ref, c_ref2, r_ref, g_ref, bt_ref, o_ref):
    o_ref[...] = _norm_block(p_ref[...], p_ref2[...], c_ref[...], c_ref2[...],
                             r_ref[...], g_ref[...], bt_ref[...])


def _comb(parts, cnts, r, gamma, beta):
    return pl.pallas_call(
        _comb_body,
        grid=(N // _BR,),
        in_specs=[
            pl.BlockSpec((1, _BR, D), lambda i: (0, i, 0)),
            pl.BlockSpec((1, _BR, D), lambda i: (1, i, 0)),
            pl.BlockSpec((1, _BR, CW), lambda i: (0, i, 0)),
            pl.BlockSpec((1, _BR, CW), lambda i: (1, i, 0)),
            pl.BlockSpec((_BR, D), lambda i: (i, 0)),
            pl.BlockSpec((1, D), lambda i: (0, 0)),
            pl.BlockSpec((1, D), lambda i: (0, 0)),
        ],
        out_specs=pl.BlockSpec((_BR, D), lambda i: (i, 0)),
        out_shape=jax.ShapeDtypeStruct((N, D), jnp.float32),
    )(parts, parts, cnts, cnts, r, gamma.reshape(1, D), beta.reshape(1, D))


def kernel(x, edge_index, W_l0, W_r0, b0, gamma0, beta0,
           W_l1, W_r1, b1, gamma1, beta1):
    src = edge_index[0].astype(jnp.int32)
    dst = edge_index[1].astype(jnp.int32)
    pad = EPAD - E
    pk = jnp.arange(pad, dtype=jnp.int32)
    src_p = jnp.concatenate([src, pk % N])
    dst_p = jnp.concatenate([dst, TRASH + pk % (NPAD - N)])

    zrow = jnp.zeros((CH, D), jnp.float32)

    a0, r0 = _lin(x, W_l0, W_r0, b0)
    ones = jnp.ones((CH, D), jnp.float32)
    cnts = _sc_cnt(dst_p, zrow, ones).reshape(NC, NPAD, D)[:, :, :CW]

    parts0 = _sc_agg(a0, src_p, dst_p, zrow).reshape(NC, NPAD, D)
    a1, r1 = _comblin(parts0, cnts, r0, gamma0, beta0, W_l1, W_r1, b1)
    parts1 = _sc_agg(a1, src_p, dst_p, zrow).reshape(NC, NPAD, D)
    return _comb(parts1, cnts, r1, gamma1, beta1)
